# baseline (speedup 1.0000x reference)
import functools

import jax
import jax.numpy as jnp
from jax import lax
from jax.experimental import pallas as pl
from jax.experimental.pallas import tpu as pltpu
from jax.experimental.pallas import tpu_sc as plsc

try:
    _INFO = plsc.get_sparse_core_info()
    _NC = _INFO.num_cores
    _NS = _INFO.num_subcores
except Exception:
    _NC, _NS = 2, 16
_NW = _NC * _NS
_LANES = 16

_CR = 96


def _add_chunk(xref, tref, CR):
    def row_body(r, c):
        for g in range(128 // _LANES):
            sl = pl.ds(g * _LANES, _LANES)
            plsc.addupdate(xref.at[r, sl], tref[r, sl])
        return c

    lax.fori_loop(0, CR, row_body, 0, unroll=4)


def _sc_add(x3, t2, B, M):
    rows_per_w = M // _NW
    n_chunks = rows_per_w // _CR
    n_pairs = n_chunks // 2

    @functools.partial(
        pl.kernel,
        mesh=plsc.VectorSubcoreMesh(core_axis_name="c", subcore_axis_name="s"),
        out_type=jax.ShapeDtypeStruct((B, M, 128), jnp.float32),
        scratch_types=(
            [pltpu.VMEM((_CR, 128), jnp.float32)] * 10
            + [pltpu.SemaphoreType.DMA] * 8
        ),
    )
    def run(x_hbm, t_hbm, o_hbm, tbuf0, tbuf1,
            xbuf0, xbuf1, xbuf2, xbuf3, xbuf4, xbuf5, xbuf6, xbuf7,
            tsem0, tsem1, xsem0, xsem1, xsem2, xsem3, osem0, osem1):
        wid = lax.axis_index("s") * _NC + lax.axis_index("c")
        base = wid * rows_per_w
        tb, tsem = (tbuf0, tbuf1), (tsem0, tsem1)
        xb = (xbuf0, xbuf1, xbuf2, xbuf3, xbuf4, xbuf5, xbuf6, xbuf7)
        xsem = (xsem0, xsem1, xsem2, xsem3)
        osem = (osem0, osem1)

        def row0(k):
            return pl.multiple_of(base + k * _CR, 8)

        def t_load(k, kp):
            return pltpu.make_async_copy(
                t_hbm.at[pl.ds(row0(k), _CR), :], tb[kp], tsem[kp])

        def x_load(k, b, kp):
            return pltpu.make_async_copy(
                x_hbm.at[b, pl.ds(row0(k), _CR), :], xb[4 * kp + b], xsem[b])

        def o_store(k, b, kp):
            return pltpu.make_async_copy(
                xb[4 * kp + b], o_hbm.at[b, pl.ds(row0(k), _CR), :],
                osem[b % 2])

        t_load(0, 0).start()
        for b in range(B):
            x_load(0, b, 0).start()

        def pair_body(kk, carry):
            for kp in range(2):
                k = kk * 2 + kp
                kq = 1 - kp
                for b in range(B):
                    if b == 0:
                        if kp == 0:
                            t_load(k + 1, 1).start()
                        else:
                            @pl.when(kk < n_pairs - 1)
                            def _():
                                t_load(k + 1, 0).start()
                        t_load(k, kp).wait()
                    if kp == 0:
                        @pl.when(kk > 0)
                        def _():
                            o_store(k - 1, b, kq).wait()
                        x_load(k + 1, b, kq).start()
                    else:
                        o_store(k - 1, b, kq).wait()

                        @pl.when(kk < n_pairs - 1)
                        def _():
                            x_load(k + 1, b, kq).start()
                    x_load(k, b, kp).wait()
                    _add_chunk(xb[4 * kp + b], tb[kp], _CR)
                    o_store(k, b, kp).start()
            return carry

        lax.fori_loop(0, n_pairs, pair_body, 0)

        for b in range(B):
            o_store(n_chunks - 1, b, 1).wait()

    return run(x3, t2)


def _to_linear_view(a):
    s, d = a.shape[-2], a.shape[-1]
    lead = a.shape[:-2]
    a5 = a.reshape(*lead, s // 8, 8, d // 128, 128)
    perm = tuple(range(len(lead))) + tuple(
        len(lead) + i for i in (0, 2, 1, 3))
    return a5.transpose(perm).reshape(*lead, s * d // 128, 128)


def _from_linear_view(a3, s, d):
    lead = a3.shape[:-2]
    a5 = a3.reshape(*lead, s // 8, d // 128, 8, 128)
    perm = tuple(range(len(lead))) + tuple(
        len(lead) + i for i in (0, 2, 1, 3))
    return a5.transpose(perm).reshape(*lead, s, d)


def kernel(x, emb_table):
    B, S, D = x.shape
    M = S * D // 128
    x3 = _to_linear_view(x)
    t2 = _to_linear_view(emb_table)
    out3 = _sc_add(x3, t2, B, M)
    return _from_linear_view(out3, S, D)


# speedup vs baseline: 1.0137x; 1.0137x over previous
import functools

import jax
import jax.numpy as jnp
from jax import lax
from jax.experimental import pallas as pl
from jax.experimental.pallas import tpu as pltpu
from jax.experimental.pallas import tpu_sc as plsc

try:
    _INFO = plsc.get_sparse_core_info()
    _NC = _INFO.num_cores
    _NS = _INFO.num_subcores
except Exception:
    _NC, _NS = 2, 16
_NW = _NC * _NS
_LANES = 16

_CR = 96


def _add_chunk(xref, tref, CR):
    def row_body(r, c):
        for g in range(128 // _LANES):
            sl = pl.ds(g * _LANES, _LANES)
            plsc.addupdate(xref.at[r, sl], tref[r, sl])
        return c

    lax.fori_loop(0, CR, row_body, 0, unroll=2)


def _sc_add(x3, t2, B, M):
    rows_per_w = M // _NW
    n_chunks = rows_per_w // _CR
    n_pairs = n_chunks // 2

    @functools.partial(
        pl.kernel,
        mesh=plsc.VectorSubcoreMesh(core_axis_name="c", subcore_axis_name="s"),
        out_type=jax.ShapeDtypeStruct((B, M, 128), jnp.float32),
        scratch_types=(
            [pltpu.VMEM((_CR, 128), jnp.float32)] * 10
            + [pltpu.SemaphoreType.DMA] * 8
        ),
    )
    def run(x_hbm, t_hbm, o_hbm, tbuf0, tbuf1,
            xbuf0, xbuf1, xbuf2, xbuf3, xbuf4, xbuf5, xbuf6, xbuf7,
            tsem0, tsem1, xsem0, xsem1, xsem2, xsem3, osem0, osem1):
        wid = lax.axis_index("s") * _NC + lax.axis_index("c")
        base = wid * rows_per_w
        tb, tsem = (tbuf0, tbuf1), (tsem0, tsem1)
        xb = (xbuf0, xbuf1, xbuf2, xbuf3, xbuf4, xbuf5, xbuf6, xbuf7)
        xsem = (xsem0, xsem1, xsem2, xsem3)
        osem = (osem0, osem1)

        def row0(k):
            return pl.multiple_of(base + k * _CR, 8)

        def t_load(k, kp):
            return pltpu.make_async_copy(
                t_hbm.at[pl.ds(row0(k), _CR), :], tb[kp], tsem[kp])

        def x_load(k, b, kp):
            return pltpu.make_async_copy(
                x_hbm.at[b, pl.ds(row0(k), _CR), :], xb[4 * kp + b], xsem[b])

        def o_store(k, b, kp):
            return pltpu.make_async_copy(
                xb[4 * kp + b], o_hbm.at[b, pl.ds(row0(k), _CR), :],
                osem[b % 2])

        t_load(0, 0).start()
        for b in range(B):
            x_load(0, b, 0).start()

        def pair_body(kk, carry):
            for kp in range(2):
                k = kk * 2 + kp
                kq = 1 - kp
                for b in range(B):
                    if b == 0:
                        if kp == 0:
                            t_load(k + 1, 1).start()
                        else:
                            @pl.when(kk < n_pairs - 1)
                            def _():
                                t_load(k + 1, 0).start()
                        t_load(k, kp).wait()
                    if kp == 0:
                        @pl.when(kk > 0)
                        def _():
                            o_store(k - 1, b, kq).wait()
                        x_load(k + 1, b, kq).start()
                    else:
                        o_store(k - 1, b, kq).wait()

                        @pl.when(kk < n_pairs - 1)
                        def _():
                            x_load(k + 1, b, kq).start()
                    x_load(k, b, kp).wait()
                    _add_chunk(xb[4 * kp + b], tb[kp], _CR)
                    o_store(k, b, kp).start()
            return carry

        lax.fori_loop(0, n_pairs, pair_body, 0)

        for b in range(B):
            o_store(n_chunks - 1, b, 1).wait()

    return run(x3, t2)


def _to_linear_view(a):
    s, d = a.shape[-2], a.shape[-1]
    lead = a.shape[:-2]
    a5 = a.reshape(*lead, s // 8, 8, d // 128, 128)
    perm = tuple(range(len(lead))) + tuple(
        len(lead) + i for i in (0, 2, 1, 3))
    return a5.transpose(perm).reshape(*lead, s * d // 128, 128)


def _from_linear_view(a3, s, d):
    lead = a3.shape[:-2]
    a5 = a3.reshape(*lead, s // 8, d // 128, 8, 128)
    perm = tuple(range(len(lead))) + tuple(
        len(lead) + i for i in (0, 2, 1, 3))
    return a5.transpose(perm).reshape(*lead, s, d)


def kernel(x, emb_table):
    B, S, D = x.shape
    M = S * D // 128
    x3 = _to_linear_view(x)
    t2 = _to_linear_view(emb_table)
    out3 = _sc_add(x3, t2, B, M)
    return _from_linear_view(out3, S, D)
